# baseline (speedup 1.0000x reference)
import functools

import jax
import jax.numpy as jnp
from jax import lax
from jax.experimental import pallas as pl
from jax.experimental.pallas import tpu as pltpu
from jax.experimental.pallas import tpu_sc as plsc

N_NODES = 10000
TEXT_D = 256
E_EDGES = 320000
MELH = 128
MELW = 64

NC, NS = 2, 16
NW = NC * NS
C_CHUNK = 128
SPW = 80
E_PAD = NW * SPW * C_CHUNK
N_PAD = 11264
ROWS_PER_SUB = N_PAD // NS

R_TILE = 400
N_TILES = N_NODES // R_TILE


def _make_deg(_MESH):
    W = 8

    @functools.partial(
        pl.kernel,
        out_type=jax.ShapeDtypeStruct((NC, N_PAD, W), jnp.float32),
        mesh=_MESH,
        scratch_types=[
            pltpu.VMEM((SPW, C_CHUNK), jnp.int32),
            pltpu.VMEM((C_CHUNK, W), jnp.float32),
            pltpu.VMEM_SHARED((N_PAD, W), jnp.float32),
        ],
        compiler_params=pltpu.CompilerParams(use_tc_tiling_on_sc=False),
    )
    def deg(dst2d_h, ones_h, zeros_h, out_h, didx, ones_v, acc):
        cid = lax.axis_index("c")
        sid = lax.axis_index("s")
        wid = cid * NS + sid
        r0 = sid * ROWS_PER_SUB
        pltpu.sync_copy(zeros_h.at[pl.ds(r0, ROWS_PER_SUB)],
                        acc.at[pl.ds(r0, ROWS_PER_SUB)])
        pltpu.sync_copy(dst2d_h.at[wid], didx)
        pltpu.sync_copy(ones_h, ones_v)
        plsc.subcore_barrier()

        def step(j, carry):
            pltpu.sync_copy(ones_v, acc.at[didx.at[j]], add=True)
            return carry

        lax.fori_loop(0, SPW, step, 0)
        plsc.subcore_barrier()
        pltpu.sync_copy(acc.at[pl.ds(r0, ROWS_PER_SUB)],
                        out_h.at[cid, pl.ds(r0, ROWS_PER_SUB)])

    return deg


def _make_agg(_MESH, W, C):
    SP = E_PAD // (NW * C)

    @functools.partial(
        pl.kernel,
        out_type=jax.ShapeDtypeStruct((NC, N_PAD, W), jnp.float32),
        mesh=_MESH,
        scratch_types=[
            pltpu.VMEM((SP, C), jnp.int32),
            pltpu.VMEM((SP, C), jnp.int32),
            pltpu.VMEM((C, W), jnp.float32),
            pltpu.VMEM((C, W), jnp.float32),
            pltpu.VMEM_SHARED((N_PAD, W), jnp.float32),
            pltpu.SemaphoreType.DMA,
            pltpu.SemaphoreType.DMA,
        ],
        compiler_params=pltpu.CompilerParams(use_tc_tiling_on_sc=False),
    )
    def agg(src2d_h, dst2d_h, table_h, zeros_h, out_h, sidx, didx,
            rows0, rows1, acc, sem0, sem1):
        cid = lax.axis_index("c")
        sid = lax.axis_index("s")
        wid = cid * NS + sid
        r0 = sid * ROWS_PER_SUB
        pltpu.sync_copy(zeros_h.at[pl.ds(r0, ROWS_PER_SUB)],
                        acc.at[pl.ds(r0, ROWS_PER_SUB)])
        pltpu.sync_copy(src2d_h.at[wid], sidx)
        pltpu.sync_copy(dst2d_h.at[wid], didx)
        plsc.subcore_barrier()

        rows = (rows0, rows1)
        sems = (sem0, sem1)
        cps = [None, None]
        cps[0] = pltpu.async_copy(table_h.at[sidx.at[0]], rows0, sem0)
        for j in range(SP):
            if j + 1 < SP:
                p = (j + 1) % 2
                cps[p] = pltpu.async_copy(
                    table_h.at[sidx.at[j + 1]], rows[p], sems[p])
            cps[j % 2].wait()
            pltpu.sync_copy(rows[j % 2], acc.at[didx.at[j]], add=True)
        plsc.subcore_barrier()
        pltpu.sync_copy(acc.at[pl.ds(r0, ROWS_PER_SUB)],
                        out_h.at[cid, pl.ds(r0, ROWS_PER_SUB)])

    return agg


@functools.lru_cache(maxsize=1)
def _sc_calls():
    mesh = plsc.VectorSubcoreMesh(core_axis_name="c", subcore_axis_name="s",
                                  num_cores=NC, num_subcores=NS)
    return _make_deg(mesh), _make_agg(mesh, 128, 64), _make_agg(mesh, 16, 128)


def _dense_body(mel_ref, text_ref, wm_ref, bm_ref, wct_ref, wcm_ref, bc_ref,
                w1_ref, d0_ref, d1_ref, h1p_ref, self1_ref, s_ref):
    bf16 = jnp.bfloat16
    mm = jnp.zeros((R_TILE, MELH), jnp.float32)
    for wb in range(MELW // 8):
        blk = jnp.swapaxes(mel_ref[:, wb * 8:(wb + 1) * 8, :], 0, 1)
        blk = blk.astype(bf16)
        for j in range(8):
            mm = mm + jnp.dot(blk[j], wm_ref[wb * 8 + j],
                              preferred_element_type=jnp.float32)
    deg = d0_ref[0][:, 0:1] + d1_ref[0][:, 0:1]
    mh = jnp.maximum(mm + bm_ref[...], 0.0)
    xb = jnp.maximum(
        jnp.dot(text_ref[...].astype(bf16), wct_ref[...],
                preferred_element_type=jnp.float32)
        + jnp.dot(mh.astype(bf16), wcm_ref[...],
                  preferred_element_type=jnp.float32)
        + bc_ref[...], 0.0)
    h1 = jnp.dot(xb.astype(bf16), w1_ref[...],
                 preferred_element_type=jnp.float32)
    s = lax.rsqrt(deg + 1.0)
    h1p_ref[...] = h1 * s
    self1_ref[...] = h1 * (s * s)
    s_ref[...] = s


def _mid_body(a0_ref, a1_ref, self1_ref, s_ref, b1_ref, w2_ref,
              h2p_ref, self2_ref):
    s = s_ref[...]
    x2 = jnp.maximum(s * (a0_ref[0] + a1_ref[0]) + self1_ref[...]
                     + b1_ref[...], 0.0)
    h2 = jnp.dot(x2, w2_ref[...], preferred_element_type=jnp.float32)
    h2p = h2 * s
    h2p_ref[...] = h2p
    self2_ref[...] = h2p * s


def _fin_body(a0_ref, a1_ref, self2_ref, s_ref, b2_ref, out_ref):
    v = (s_ref[...] * (a0_ref[0] + a1_ref[0])
         + self2_ref[...] + b2_ref[...])
    out_ref[...] = v[:, :4]


def _row_spec(w):
    return pl.BlockSpec((R_TILE, w), lambda i: (i, 0))


def _full_spec(r, w):
    return pl.BlockSpec((r, w), lambda i: (0, 0))


def _part_spec(w, c):
    return pl.BlockSpec((1, R_TILE, w), lambda i, c=c: (c, i, 0))


_dense_call = pl.pallas_call(
    _dense_body,
    grid=(N_TILES,),
    in_specs=[
        pl.BlockSpec((R_TILE, MELW, MELH), lambda i: (i, 0, 0)),
        _row_spec(TEXT_D),
        pl.BlockSpec((MELW, MELH, 128), lambda i: (0, 0, 0)),
        _full_spec(1, 128),
        _full_spec(TEXT_D, 128),
        _full_spec(128, 128),
        _full_spec(1, 128),
        _full_spec(128, 128),
        _part_spec(8, 0),
        _part_spec(8, 1),
    ],
    out_specs=[_row_spec(128), _row_spec(128), _row_spec(1)],
    out_shape=[
        jax.ShapeDtypeStruct((N_NODES, 128), jnp.float32),
        jax.ShapeDtypeStruct((N_NODES, 128), jnp.float32),
        jax.ShapeDtypeStruct((N_NODES, 1), jnp.float32),
    ],
)

R_MID = 1000
R_FIN = 2000

_mid_call = pl.pallas_call(
    _mid_body,
    grid=(N_NODES // R_MID,),
    in_specs=[
        pl.BlockSpec((1, R_MID, 128), lambda i: (0, i, 0)),
        pl.BlockSpec((1, R_MID, 128), lambda i: (1, i, 0)),
        pl.BlockSpec((R_MID, 128), lambda i: (i, 0)),
        pl.BlockSpec((R_MID, 1), lambda i: (i, 0)),
        _full_spec(1, 128), _full_spec(128, 16),
    ],
    out_specs=[pl.BlockSpec((R_MID, 16), lambda i: (i, 0)),
               pl.BlockSpec((R_MID, 16), lambda i: (i, 0))],
    out_shape=[
        jax.ShapeDtypeStruct((N_NODES, 16), jnp.float32),
        jax.ShapeDtypeStruct((N_NODES, 16), jnp.float32),
    ],
)

_fin_call = pl.pallas_call(
    _fin_body,
    grid=(N_NODES // R_FIN,),
    in_specs=[
        pl.BlockSpec((1, R_FIN, 16), lambda i: (0, i, 0)),
        pl.BlockSpec((1, R_FIN, 16), lambda i: (1, i, 0)),
        pl.BlockSpec((R_FIN, 16), lambda i: (i, 0)),
        pl.BlockSpec((R_FIN, 1), lambda i: (i, 0)),
        _full_spec(1, 16),
    ],
    out_specs=pl.BlockSpec((R_FIN, 4), lambda i: (i, 0)),
    out_shape=jax.ShapeDtypeStruct((N_NODES, 4), jnp.float32),
)


def kernel(text, mel, edge_index, W_mel, b_mel, W_cat, b_cat, W1, b1, W2, b2):
    f32 = jnp.float32
    mel3 = mel.transpose(0, 2, 1)
    bf16 = jnp.bfloat16
    wm_p = W_mel.reshape(MELH, MELW, -1).transpose(1, 0, 2).astype(bf16)
    src = edge_index[0]
    dst = edge_index[1]
    pad = E_PAD - E_EDGES
    src_f = jnp.concatenate(
        [src, jnp.arange(pad, dtype=src.dtype) % N_NODES])
    dst_f = jnp.concatenate(
        [dst, N_NODES + (jnp.arange(pad, dtype=dst.dtype)
                         % (N_PAD - N_NODES))])
    src_p = src_f.reshape(NW, SPW, C_CHUNK)
    dst_p = dst_f.reshape(NW, SPW, C_CHUNK)
    src_p64 = src_f.reshape(NW, 2 * SPW, C_CHUNK // 2)
    dst_p64 = dst_f.reshape(NW, 2 * SPW, C_CHUNK // 2)

    _deg_call, _agg128_call, _agg16_call = _sc_calls()
    ones_src = jnp.ones((C_CHUNK, 8), f32)
    z8 = jnp.zeros((N_PAD, 8), f32)
    z16 = jnp.zeros((N_PAD, 16), f32)
    z128 = jnp.zeros((N_PAD, 128), f32)

    degp = _deg_call(dst_p, ones_src, z8)

    h1p, self1, s = _dense_call(
        mel3, text, wm_p, b_mel.reshape(1, -1), W_cat[:TEXT_D].astype(bf16),
        W_cat[TEXT_D:].astype(bf16), b_cat.reshape(1, -1), W1.astype(bf16),
        degp, degp)

    agg1 = _agg128_call(src_p64, dst_p64, h1p, z128)

    W2p = jnp.pad(W2, ((0, 0), (0, 12)))
    h2p, self2 = _mid_call(agg1, agg1, self1, s, b1.reshape(1, -1), W2p)

    agg2 = _agg16_call(src_p, dst_p, h2p, z16)

    b2p = jnp.pad(b2, (0, 12)).reshape(1, 16)
    return _fin_call(agg2, agg2, self2, s, b2p)


# speedup vs baseline: 1.0633x; 1.0633x over previous
import functools

import jax
import jax.numpy as jnp
from jax import lax
from jax.experimental import pallas as pl
from jax.experimental.pallas import tpu as pltpu
from jax.experimental.pallas import tpu_sc as plsc

N_NODES = 10000
TEXT_D = 256
E_EDGES = 320000
MELH = 128
MELW = 64

NC, NS = 2, 16
NW = NC * NS
C_CHUNK = 128
SPW = 80
E_PAD = NW * SPW * C_CHUNK
N_PAD = 11264
ROWS_PER_SUB = N_PAD // NS

R_TILE = 400
N_TILES = N_NODES // R_TILE


def _make_deg(_MESH):
    W = 8

    @functools.partial(
        pl.kernel,
        out_type=jax.ShapeDtypeStruct((NC, N_PAD, W), jnp.float32),
        mesh=_MESH,
        scratch_types=[
            pltpu.VMEM((SPW, C_CHUNK), jnp.int32),
            pltpu.VMEM((C_CHUNK, W), jnp.float32),
            pltpu.VMEM_SHARED((N_PAD, W), jnp.float32),
        ],
        compiler_params=pltpu.CompilerParams(use_tc_tiling_on_sc=False),
    )
    def deg(dst2d_h, ones_h, zeros_h, out_h, didx, ones_v, acc):
        cid = lax.axis_index("c")
        sid = lax.axis_index("s")
        wid = cid * NS + sid
        r0 = sid * ROWS_PER_SUB
        pltpu.sync_copy(zeros_h.at[pl.ds(r0, ROWS_PER_SUB)],
                        acc.at[pl.ds(r0, ROWS_PER_SUB)])
        pltpu.sync_copy(dst2d_h.at[wid], didx)
        pltpu.sync_copy(ones_h, ones_v)
        plsc.subcore_barrier()

        def step(j, carry):
            pltpu.sync_copy(ones_v, acc.at[didx.at[j]], add=True)
            return carry

        lax.fori_loop(0, SPW, step, 0)
        plsc.subcore_barrier()
        pltpu.sync_copy(acc.at[pl.ds(r0, ROWS_PER_SUB)],
                        out_h.at[cid, pl.ds(r0, ROWS_PER_SUB)])

    return deg


def _make_agg(_MESH, W, C):
    SP = E_PAD // (NW * C)

    @functools.partial(
        pl.kernel,
        out_type=jax.ShapeDtypeStruct((NC, N_PAD, W), jnp.float32),
        mesh=_MESH,
        scratch_types=[
            pltpu.VMEM((SP, C), jnp.int32),
            pltpu.VMEM((SP, C), jnp.int32),
            pltpu.VMEM((C, W), jnp.float32),
            pltpu.VMEM((C, W), jnp.float32),
            pltpu.VMEM_SHARED((N_PAD, W), jnp.float32),
            pltpu.SemaphoreType.DMA,
            pltpu.SemaphoreType.DMA,
        ],
        compiler_params=pltpu.CompilerParams(use_tc_tiling_on_sc=False),
    )
    def agg(src2d_h, dst2d_h, table_h, zeros_h, out_h, sidx, didx,
            rows0, rows1, acc, sem0, sem1):
        cid = lax.axis_index("c")
        sid = lax.axis_index("s")
        wid = cid * NS + sid
        r0 = sid * ROWS_PER_SUB
        pltpu.sync_copy(zeros_h.at[pl.ds(r0, ROWS_PER_SUB)],
                        acc.at[pl.ds(r0, ROWS_PER_SUB)])
        pltpu.sync_copy(src2d_h.at[wid], sidx)
        pltpu.sync_copy(dst2d_h.at[wid], didx)
        plsc.subcore_barrier()

        rows = (rows0, rows1)
        sems = (sem0, sem1)
        cps = [None, None]
        cps[0] = pltpu.async_copy(table_h.at[sidx.at[0]], rows0, sem0)
        for j in range(SP):
            if j + 1 < SP:
                p = (j + 1) % 2
                cps[p] = pltpu.async_copy(
                    table_h.at[sidx.at[j + 1]], rows[p], sems[p])
            cps[j % 2].wait()
            pltpu.sync_copy(rows[j % 2], acc.at[didx.at[j]], add=True)
        plsc.subcore_barrier()
        pltpu.sync_copy(acc.at[pl.ds(r0, ROWS_PER_SUB)],
                        out_h.at[cid, pl.ds(r0, ROWS_PER_SUB)])

    return agg


@functools.lru_cache(maxsize=1)
def _sc_calls():
    mesh = plsc.VectorSubcoreMesh(core_axis_name="c", subcore_axis_name="s",
                                  num_cores=NC, num_subcores=NS)
    return _make_deg(mesh), _make_agg(mesh, 128, 64), _make_agg(mesh, 16, 128)


def _dense_body(mel_ref, text_ref, wm_ref, bm_ref, wct_ref, wcm_ref, bc_ref,
                w1_ref, d0_ref, d1_ref, h1p_ref, self1_ref, s_ref):
    mm = jnp.zeros((R_TILE, MELH), jnp.float32)
    for wb in range(MELW // 8):
        blk = jnp.swapaxes(mel_ref[:, wb * 8:(wb + 1) * 8, :], 0, 1)
        for j in range(8):
            mm = mm + jnp.dot(blk[j], wm_ref[wb * 8 + j],
                              preferred_element_type=jnp.float32)
    deg = d0_ref[0][:, 0:1] + d1_ref[0][:, 0:1]
    mh = jnp.maximum(mm + bm_ref[...], 0.0)
    xb = jnp.maximum(
        jnp.dot(text_ref[...], wct_ref[...],
                preferred_element_type=jnp.float32)
        + jnp.dot(mh, wcm_ref[...], preferred_element_type=jnp.float32)
        + bc_ref[...], 0.0)
    h1 = jnp.dot(xb, w1_ref[...], preferred_element_type=jnp.float32)
    s = lax.rsqrt(deg + 1.0)
    h1p_ref[...] = h1 * s
    self1_ref[...] = h1 * (s * s)
    s_ref[...] = s


def _mid_body(a0_ref, a1_ref, self1_ref, s_ref, b1_ref, w2_ref,
              h2p_ref, self2_ref):
    s = s_ref[...]
    x2 = jnp.maximum(s * (a0_ref[0] + a1_ref[0]) + self1_ref[...]
                     + b1_ref[...], 0.0)
    h2 = jnp.dot(x2, w2_ref[...], preferred_element_type=jnp.float32)
    h2p = h2 * s
    h2p_ref[...] = h2p
    self2_ref[...] = h2p * s


def _fin_body(a0_ref, a1_ref, self2_ref, s_ref, b2_ref, out_ref):
    v = (s_ref[...] * (a0_ref[0] + a1_ref[0])
         + self2_ref[...] + b2_ref[...])
    out_ref[...] = v[:, :4]


def _row_spec(w):
    return pl.BlockSpec((R_TILE, w), lambda i: (i, 0))


def _full_spec(r, w):
    return pl.BlockSpec((r, w), lambda i: (0, 0))


def _part_spec(w, c):
    return pl.BlockSpec((1, R_TILE, w), lambda i, c=c: (c, i, 0))


_dense_call = pl.pallas_call(
    _dense_body,
    grid=(N_TILES,),
    in_specs=[
        pl.BlockSpec((R_TILE, MELW, MELH), lambda i: (i, 0, 0)),
        _row_spec(TEXT_D),
        pl.BlockSpec((MELW, MELH, 128), lambda i: (0, 0, 0)),
        _full_spec(1, 128),
        _full_spec(TEXT_D, 128),
        _full_spec(128, 128),
        _full_spec(1, 128),
        _full_spec(128, 128),
        _part_spec(8, 0),
        _part_spec(8, 1),
    ],
    out_specs=[_row_spec(128), _row_spec(128), _row_spec(1)],
    out_shape=[
        jax.ShapeDtypeStruct((N_NODES, 128), jnp.float32),
        jax.ShapeDtypeStruct((N_NODES, 128), jnp.float32),
        jax.ShapeDtypeStruct((N_NODES, 1), jnp.float32),
    ],
)

R_MID = 1000
R_FIN = 2000

_mid_call = pl.pallas_call(
    _mid_body,
    grid=(N_NODES // R_MID,),
    in_specs=[
        pl.BlockSpec((1, R_MID, 128), lambda i: (0, i, 0)),
        pl.BlockSpec((1, R_MID, 128), lambda i: (1, i, 0)),
        pl.BlockSpec((R_MID, 128), lambda i: (i, 0)),
        pl.BlockSpec((R_MID, 1), lambda i: (i, 0)),
        _full_spec(1, 128), _full_spec(128, 16),
    ],
    out_specs=[pl.BlockSpec((R_MID, 16), lambda i: (i, 0)),
               pl.BlockSpec((R_MID, 16), lambda i: (i, 0))],
    out_shape=[
        jax.ShapeDtypeStruct((N_NODES, 16), jnp.float32),
        jax.ShapeDtypeStruct((N_NODES, 16), jnp.float32),
    ],
)

_fin_call = pl.pallas_call(
    _fin_body,
    grid=(N_NODES // R_FIN,),
    in_specs=[
        pl.BlockSpec((1, R_FIN, 16), lambda i: (0, i, 0)),
        pl.BlockSpec((1, R_FIN, 16), lambda i: (1, i, 0)),
        pl.BlockSpec((R_FIN, 16), lambda i: (i, 0)),
        pl.BlockSpec((R_FIN, 1), lambda i: (i, 0)),
        _full_spec(1, 16),
    ],
    out_specs=pl.BlockSpec((R_FIN, 4), lambda i: (i, 0)),
    out_shape=jax.ShapeDtypeStruct((N_NODES, 4), jnp.float32),
)


def kernel(text, mel, edge_index, W_mel, b_mel, W_cat, b_cat, W1, b1, W2, b2):
    f32 = jnp.float32
    mel3 = mel.transpose(0, 2, 1)
    wm_p = W_mel.reshape(MELH, MELW, -1).transpose(1, 0, 2)
    src = edge_index[0]
    dst = edge_index[1]
    pad = E_PAD - E_EDGES
    src_f = jnp.concatenate(
        [src, jnp.arange(pad, dtype=src.dtype) % N_NODES])
    dst_f = jnp.concatenate(
        [dst, N_NODES + (jnp.arange(pad, dtype=dst.dtype)
                         % (N_PAD - N_NODES))])
    src_p = src_f.reshape(NW, SPW, C_CHUNK)
    dst_p = dst_f.reshape(NW, SPW, C_CHUNK)
    src_p64 = src_f.reshape(NW, 2 * SPW, C_CHUNK // 2)
    dst_p64 = dst_f.reshape(NW, 2 * SPW, C_CHUNK // 2)

    _deg_call, _agg128_call, _agg16_call = _sc_calls()
    ones_src = jnp.ones((C_CHUNK, 8), f32)
    z8 = jnp.zeros((N_PAD, 8), f32)
    z16 = jnp.zeros((N_PAD, 16), f32)
    z128 = jnp.zeros((N_PAD, 128), f32)

    degp = _deg_call(dst_p, ones_src, z8)

    h1p, self1, s = _dense_call(
        mel3, text, wm_p, b_mel.reshape(1, -1), W_cat[:TEXT_D],
        W_cat[TEXT_D:], b_cat.reshape(1, -1), W1, degp, degp)

    agg1 = _agg128_call(src_p64, dst_p64, h1p, z128)

    W2p = jnp.pad(W2, ((0, 0), (0, 12)))
    h2p, self2 = _mid_call(agg1, agg1, self1, s, b1.reshape(1, -1), W2p)

    agg2 = _agg16_call(src_p, dst_p, h2p, z16)

    b2p = jnp.pad(b2, (0, 12)).reshape(1, 16)
    return _fin_call(agg2, agg2, self2, s, b2p)


# speedup vs baseline: 1.0743x; 1.0103x over previous
import functools

import jax
import jax.numpy as jnp
from jax import lax
from jax.experimental import pallas as pl
from jax.experimental.pallas import tpu as pltpu
from jax.experimental.pallas import tpu_sc as plsc

N_NODES = 10000
TEXT_D = 256
E_EDGES = 320000
MELH = 128
MELW = 64

NC, NS = 2, 16
NW = NC * NS
C_CHUNK = 128
SPW = 80
E_PAD = NW * SPW * C_CHUNK
N_PAD = 11264
ROWS_PER_SUB = N_PAD // NS

R_TILE = 400
N_TILES = N_NODES // R_TILE


def _make_deg(_MESH):
    W = 8

    @functools.partial(
        pl.kernel,
        out_type=jax.ShapeDtypeStruct((NC, N_PAD, W), jnp.float32),
        mesh=_MESH,
        scratch_types=[
            pltpu.VMEM((SPW, C_CHUNK), jnp.int32),
            pltpu.VMEM((C_CHUNK, W), jnp.float32),
            pltpu.VMEM_SHARED((N_PAD, W), jnp.float32),
        ],
        compiler_params=pltpu.CompilerParams(use_tc_tiling_on_sc=False),
    )
    def deg(ed_h, ones_h, zeros_h, out_h, didx, ones_v, acc):
        cid = lax.axis_index("c")
        sid = lax.axis_index("s")
        wid = cid * NS + sid
        r0 = sid * ROWS_PER_SUB
        pltpu.sync_copy(zeros_h.at[pl.ds(r0, ROWS_PER_SUB)],
                        acc.at[pl.ds(r0, ROWS_PER_SUB)])
        pltpu.sync_copy(ed_h.at[1, wid], didx)
        pltpu.sync_copy(ones_h, ones_v)
        plsc.subcore_barrier()

        def step(j, carry):
            pltpu.sync_copy(ones_v, acc.at[didx.at[j]], add=True)
            return carry

        lax.fori_loop(0, SPW, step, 0)
        plsc.subcore_barrier()
        pltpu.sync_copy(acc.at[pl.ds(r0, ROWS_PER_SUB)],
                        out_h.at[cid, pl.ds(r0, ROWS_PER_SUB)])

    return deg


def _make_agg(_MESH, W, C):
    SP = E_PAD // (NW * C)

    @functools.partial(
        pl.kernel,
        out_type=jax.ShapeDtypeStruct((NC, N_PAD, W), jnp.float32),
        mesh=_MESH,
        scratch_types=[
            pltpu.VMEM((SP, C), jnp.int32),
            pltpu.VMEM((SP, C), jnp.int32),
            pltpu.VMEM((C, W), jnp.float32),
            pltpu.VMEM((C, W), jnp.float32),
            pltpu.VMEM_SHARED((N_PAD, W), jnp.float32),
            pltpu.SemaphoreType.DMA,
            pltpu.SemaphoreType.DMA,
        ],
        compiler_params=pltpu.CompilerParams(use_tc_tiling_on_sc=False),
    )
    def agg(ed_h, table_h, zeros_h, out_h, sidx, didx,
            rows0, rows1, acc, sem0, sem1):
        cid = lax.axis_index("c")
        sid = lax.axis_index("s")
        wid = cid * NS + sid
        r0 = sid * ROWS_PER_SUB
        pltpu.sync_copy(zeros_h.at[pl.ds(r0, ROWS_PER_SUB)],
                        acc.at[pl.ds(r0, ROWS_PER_SUB)])
        pltpu.sync_copy(ed_h.at[0, wid], sidx)
        pltpu.sync_copy(ed_h.at[1, wid], didx)
        plsc.subcore_barrier()

        rows = (rows0, rows1)
        sems = (sem0, sem1)
        cps = [None, None]
        cps[0] = pltpu.async_copy(table_h.at[sidx.at[0]], rows0, sem0)
        for j in range(SP):
            if j + 1 < SP:
                p = (j + 1) % 2
                cps[p] = pltpu.async_copy(
                    table_h.at[sidx.at[j + 1]], rows[p], sems[p])
            cps[j % 2].wait()
            pltpu.sync_copy(rows[j % 2], acc.at[didx.at[j]], add=True)
        plsc.subcore_barrier()
        pltpu.sync_copy(acc.at[pl.ds(r0, ROWS_PER_SUB)],
                        out_h.at[cid, pl.ds(r0, ROWS_PER_SUB)])

    return agg


@functools.lru_cache(maxsize=1)
def _sc_calls():
    mesh = plsc.VectorSubcoreMesh(core_axis_name="c", subcore_axis_name="s",
                                  num_cores=NC, num_subcores=NS)
    return _make_deg(mesh), _make_agg(mesh, 128, 64), _make_agg(mesh, 16, 128)


def _dense_body(mel_ref, text_ref, wm_ref, bm_ref, wct_ref, wcm_ref, bc_ref,
                w1_ref, d0_ref, d1_ref, h1p_ref, self1_ref, s_ref):
    mm = jnp.zeros((R_TILE, MELH), jnp.float32)
    for wb in range(MELW // 8):
        blk = jnp.swapaxes(mel_ref[:, wb * 8:(wb + 1) * 8, :], 0, 1)
        for j in range(8):
            mm = mm + jnp.dot(blk[j], wm_ref[wb * 8 + j],
                              preferred_element_type=jnp.float32)
    deg = d0_ref[0][:, 0:1] + d1_ref[0][:, 0:1]
    mh = jnp.maximum(mm + bm_ref[...], 0.0)
    xb = jnp.maximum(
        jnp.dot(text_ref[...], wct_ref[...],
                preferred_element_type=jnp.float32)
        + jnp.dot(mh, wcm_ref[...], preferred_element_type=jnp.float32)
        + bc_ref[...], 0.0)
    h1 = jnp.dot(xb, w1_ref[...], preferred_element_type=jnp.float32)
    s = lax.rsqrt(deg + 1.0)
    h1p_ref[...] = h1 * s
    self1_ref[...] = h1 * (s * s)
    s_ref[...] = s


def _mid_body(a0_ref, a1_ref, self1_ref, s_ref, b1_ref, w2_ref,
              h2p_ref, self2_ref):
    s = s_ref[...]
    x2 = jnp.maximum(s * (a0_ref[0] + a1_ref[0]) + self1_ref[...]
                     + b1_ref[...], 0.0)
    h2 = jnp.dot(x2, w2_ref[...], preferred_element_type=jnp.float32)
    h2p = h2 * s
    h2p_ref[...] = h2p
    self2_ref[...] = h2p * s


def _fin_body(a0_ref, a1_ref, self2_ref, s_ref, b2_ref, out_ref):
    v = (s_ref[...] * (a0_ref[0] + a1_ref[0])
         + self2_ref[...] + b2_ref[...])
    out_ref[...] = v[:, :4]


def _row_spec(w):
    return pl.BlockSpec((R_TILE, w), lambda i: (i, 0))


def _full_spec(r, w):
    return pl.BlockSpec((r, w), lambda i: (0, 0))


def _part_spec(w, c):
    return pl.BlockSpec((1, R_TILE, w), lambda i, c=c: (c, i, 0))


_dense_call = pl.pallas_call(
    _dense_body,
    grid=(N_TILES,),
    in_specs=[
        pl.BlockSpec((R_TILE, MELW, MELH), lambda i: (i, 0, 0)),
        _row_spec(TEXT_D),
        pl.BlockSpec((MELW, MELH, 128), lambda i: (0, 0, 0)),
        _full_spec(1, 128),
        _full_spec(TEXT_D, 128),
        _full_spec(128, 128),
        _full_spec(1, 128),
        _full_spec(128, 128),
        _part_spec(8, 0),
        _part_spec(8, 1),
    ],
    out_specs=[_row_spec(128), _row_spec(128), _row_spec(1)],
    out_shape=[
        jax.ShapeDtypeStruct((N_NODES, 128), jnp.float32),
        jax.ShapeDtypeStruct((N_NODES, 128), jnp.float32),
        jax.ShapeDtypeStruct((N_NODES, 1), jnp.float32),
    ],
)

R_MID = 1000
R_FIN = 2000

_mid_call = pl.pallas_call(
    _mid_body,
    grid=(N_NODES // R_MID,),
    in_specs=[
        pl.BlockSpec((1, R_MID, 128), lambda i: (0, i, 0)),
        pl.BlockSpec((1, R_MID, 128), lambda i: (1, i, 0)),
        pl.BlockSpec((R_MID, 128), lambda i: (i, 0)),
        pl.BlockSpec((R_MID, 1), lambda i: (i, 0)),
        _full_spec(1, 128), _full_spec(128, 16),
    ],
    out_specs=[pl.BlockSpec((R_MID, 16), lambda i: (i, 0)),
               pl.BlockSpec((R_MID, 16), lambda i: (i, 0))],
    out_shape=[
        jax.ShapeDtypeStruct((N_NODES, 16), jnp.float32),
        jax.ShapeDtypeStruct((N_NODES, 16), jnp.float32),
    ],
)

_fin_call = pl.pallas_call(
    _fin_body,
    grid=(N_NODES // R_FIN,),
    in_specs=[
        pl.BlockSpec((1, R_FIN, 16), lambda i: (0, i, 0)),
        pl.BlockSpec((1, R_FIN, 16), lambda i: (1, i, 0)),
        pl.BlockSpec((R_FIN, 16), lambda i: (i, 0)),
        pl.BlockSpec((R_FIN, 1), lambda i: (i, 0)),
        _full_spec(1, 16),
    ],
    out_specs=pl.BlockSpec((R_FIN, 4), lambda i: (i, 0)),
    out_shape=jax.ShapeDtypeStruct((N_NODES, 4), jnp.float32),
)


def kernel(text, mel, edge_index, W_mel, b_mel, W_cat, b_cat, W1, b1, W2, b2):
    f32 = jnp.float32
    mel3 = mel.transpose(0, 2, 1)
    wm_p = W_mel.reshape(MELH, MELW, -1).transpose(1, 0, 2)
    pad = E_PAD - E_EDGES
    ar = jnp.arange(pad, dtype=edge_index.dtype)
    ed_f = jnp.concatenate(
        [edge_index,
         jnp.stack([ar % N_NODES, N_NODES + (ar % (N_PAD - N_NODES))])],
        axis=1)
    ed_p = ed_f.reshape(2, NW, SPW, C_CHUNK)
    ed_p64 = ed_f.reshape(2, NW, 2 * SPW, C_CHUNK // 2)

    _deg_call, _agg128_call, _agg16_call = _sc_calls()
    ones_src = jnp.ones((C_CHUNK, 8), f32)
    z8 = jnp.zeros((N_PAD, 8), f32)
    z16 = jnp.zeros((N_PAD, 16), f32)
    z128 = jnp.zeros((N_PAD, 128), f32)

    degp = _deg_call(ed_p, ones_src, z8)

    h1p, self1, s = _dense_call(
        mel3, text, wm_p, b_mel.reshape(1, -1), W_cat[:TEXT_D],
        W_cat[TEXT_D:], b_cat.reshape(1, -1), W1, degp, degp)

    agg1 = _agg128_call(ed_p64, h1p, z128)

    W2p = jnp.pad(W2, ((0, 0), (0, 12)))
    h2p, self2 = _mid_call(agg1, agg1, self1, s, b1.reshape(1, -1), W2p)

    agg2 = _agg16_call(ed_p, h2p, z16)

    b2p = jnp.pad(b2, (0, 12)).reshape(1, 16)
    return _fin_call(agg2, agg2, self2, s, b2p)
